# Initial kernel scaffold; baseline (speedup 1.0000x reference)
import jax
import jax.numpy as jnp
from jax.experimental import pallas as pl


def kernel(x, Wg, We):
    raise NotImplementedError("write your pallas kernel here")


# speedup vs baseline: 5.0732x; 5.0732x over previous
import functools

import jax
import jax.numpy as jnp
from jax.experimental import pallas as pl
from jax.experimental.pallas import tpu as pltpu

N_TOK = 16384
D_IN = 2048
G_GRP = 8
E_PER_G = 8
E_TOT = G_GRP * E_PER_G
W_COLS = 2 * E_TOT
BLK = 1024


def _router_block(x_ref, w_ref, m_ref, mask_ref, nw_ref):
    z = jnp.dot(x_ref[...], w_ref[...], preferred_element_type=jnp.float32,
                precision=jax.lax.Precision.DEFAULT)
    e = jnp.exp(z)
    sums = jnp.dot(e, m_ref[...], preferred_element_type=jnp.float32,
                   precision=jax.lax.Precision.HIGHEST)
    probs = e / sums
    thr = probs >= 0.125
    ep = probs[:, :E_TOT]
    gp = probs[:, E_TOT:]
    valid = thr[:, :E_TOT] & thr[:, E_TOT:]
    sel = jnp.where(valid, ep * gp, 0.0)
    wsum = jnp.dot(sel, jnp.ones((E_TOT, E_TOT), jnp.float32),
                   preferred_element_type=jnp.float32,
                   precision=jax.lax.Precision.HIGHEST)
    wsum = jnp.maximum(wsum, 1e-9)
    mask_ref[...] = valid.astype(jnp.int32)
    nw_ref[...] = sel / wsum


@functools.partial(jax.jit, static_argnames=())
def kernel(x, Wg, We):
    n = x.shape[0]
    wg_rep = jnp.repeat(Wg, E_PER_G, axis=0)
    wct = jnp.concatenate([We, wg_rep], axis=0).T
    r = jax.lax.broadcasted_iota(jnp.int32, (W_COLS, W_COLS), 0)
    c = jax.lax.broadcasted_iota(jnp.int32, (W_COLS, W_COLS), 1)
    lower = (r < E_TOT) & (c < E_TOT) & ((r // E_PER_G) == (c // E_PER_G))
    upper = (r >= E_TOT) & (c >= E_TOT)
    m = jnp.where(lower, 1.0, 0.0) + jnp.where(upper, 0.125, 0.0)
    m = m.astype(jnp.float32)

    grid = (n // BLK,)
    mask_i32, nw = pl.pallas_call(
        _router_block,
        grid=grid,
        in_specs=[
            pl.BlockSpec((BLK, D_IN), lambda i: (i, 0)),
            pl.BlockSpec((D_IN, W_COLS), lambda i: (0, 0)),
            pl.BlockSpec((W_COLS, W_COLS), lambda i: (0, 0)),
        ],
        out_specs=[
            pl.BlockSpec((BLK, E_TOT), lambda i: (i, 0)),
            pl.BlockSpec((BLK, E_TOT), lambda i: (i, 0)),
        ],
        out_shape=[
            jax.ShapeDtypeStruct((n, E_TOT), jnp.int32),
            jax.ShapeDtypeStruct((n, E_TOT), jnp.float32),
        ],
    )(x, wct, m)
    return mask_i32.astype(jnp.bool_), nw
